# baseline (speedup 1.0000x reference)
import functools

import jax
import jax.numpy as jnp
from jax.experimental import pallas as pl
from jax.experimental.pallas import tpu as pltpu

_HID = 32


def _mlp_plane_body(d_ref, a_ref, W1_ref, b1_ref, W2_ref, b2_ref, y_ref, *, din):
    o = [None] * 9
    for k in range(_HID):
        h = d_ref[0] * W1_ref[0, k]
        for j in range(1, din):
            h = h + d_ref[j] * W1_ref[j, k]
        h = jnp.maximum(h + b1_ref[k], 0.0)
        for r in range(9):
            if k == 0:
                o[r] = h * W2_ref[k, r] + b2_ref[r]
            else:
                o[r] = o[r] + h * W2_ref[k, r]
    a0, a1, a2 = a_ref[0], a_ref[1], a_ref[2]
    for i in range(3):
        y_ref[i] = o[3 * i] * a0 + o[3 * i + 1] * a1 + o[3 * i + 2] * a2


def _edge_mlp_planes(dt, at, W1, b1, W2, b2, br):
    din, m, _ = dt.shape
    assert m % br == 0, (m, br)
    grid = m // br
    return pl.pallas_call(
        functools.partial(_mlp_plane_body, din=din),
        grid=(grid,),
        in_specs=[
            pl.BlockSpec((din, br, 128), lambda i: (0, i, 0)),
            pl.BlockSpec((3, br, 128), lambda i: (0, i, 0)),
            pl.BlockSpec(memory_space=pltpu.SMEM),
            pl.BlockSpec(memory_space=pltpu.SMEM),
            pl.BlockSpec(memory_space=pltpu.SMEM),
            pl.BlockSpec(memory_space=pltpu.SMEM),
        ],
        out_specs=pl.BlockSpec((3, br, 128), lambda i: (0, i, 0)),
        out_shape=jax.ShapeDtypeStruct((3, m, 128), jnp.float32),
    )(dt, at, W1, b1, W2, b2)


def _term(xt, edges, attr, W1, b1, W2, b2, n):
    e = attr.shape[0]
    ep = -(-e // 1024) * 1024
    m = ep // 128
    br = next(c for c in (64, 56, 32, 16, 8) if m % c == 0)
    if edges.shape[0] == 2:
        d = jnp.take(xt, edges[0], axis=1) - jnp.take(xt, edges[1], axis=1)
        dst = edges[1]
    else:
        xj = jnp.take(xt, edges[0], axis=1)
        xk = jnp.take(xt, edges[1], axis=1)
        xi = jnp.take(xt, edges[2], axis=1)
        d = jnp.concatenate((xk - xj, xi - xk), axis=0)
        dst = edges[2]
    a = attr.T
    d = jnp.pad(d, ((0, 0), (0, ep - e)))
    a = jnp.pad(a, ((0, 0), (0, ep - e)))
    yt = _edge_mlp_planes(d.reshape(-1, m, 128), a.reshape(3, m, 128),
                          W1, b1, W2, b2, br)
    y = yt.reshape(3, ep)[:, :e].T
    return jax.ops.segment_sum(y, dst, num_segments=n)


def kernel(x, edge_2body, edge_3body, edge_2bodySelf, edge_1body,
           edge_attr_2body, edge_attr_3body, edge_attr_2bodySelf, edge_attr_1body,
           W1_2b, b1_2b, W2_2b, b2_2b,
           W1_3b, b1_3b, W2_3b, b2_3b,
           W1_s, b1_s, W2_s, b2_s):
    n = x.shape[0]
    xt = x.T
    v2 = _term(xt, edge_2body, edge_attr_2body, W1_2b, b1_2b, W2_2b, b2_2b, n)
    v3 = _term(xt, edge_3body, edge_attr_3body, W1_3b, b1_3b, W2_3b, b2_3b, n)
    vs = _term(xt, edge_2bodySelf, edge_attr_2bodySelf, W1_s, b1_s, W2_s, b2_s, n)
    return v2 + v3 + vs


# speedup vs baseline: 1.4734x; 1.4734x over previous
import functools

import jax
import jax.numpy as jnp
from jax import lax
from jax.experimental import pallas as pl
from jax.experimental.pallas import tpu as pltpu
from jax.experimental.pallas import tpu_sc as plsc

_HID = 32
_NPAD = 50176
_WSC = 12512


def _mlp_plane_body(d_ref, a_ref, W1_ref, b1_ref, W2_ref, b2_ref, y_ref, *, din):
    o = [None] * 9
    for k in range(_HID):
        h = d_ref[0] * W1_ref[0, k]
        for j in range(1, din):
            h = h + d_ref[j] * W1_ref[j, k]
        h = jnp.maximum(h + b1_ref[k], 0.0)
        for r in range(9):
            if k == 0:
                o[r] = h * W2_ref[k, r] + b2_ref[r]
            else:
                o[r] = o[r] + h * W2_ref[k, r]
    a0, a1, a2 = a_ref[0], a_ref[1], a_ref[2]
    for i in range(3):
        y_ref[i] = o[3 * i] * a0 + o[3 * i + 1] * a1 + o[3 * i + 2] * a2


def _edge_mlp_planes(dt, at, W1, b1, W2, b2, br):
    din, m, _ = dt.shape
    assert m % br == 0, (m, br)
    grid = m // br
    return pl.pallas_call(
        functools.partial(_mlp_plane_body, din=din),
        grid=(grid,),
        in_specs=[
            pl.BlockSpec((din, br, 128), lambda i: (0, i, 0)),
            pl.BlockSpec((3, br, 128), lambda i: (0, i, 0)),
            pl.BlockSpec(memory_space=pltpu.SMEM),
            pl.BlockSpec(memory_space=pltpu.SMEM),
            pl.BlockSpec(memory_space=pltpu.SMEM),
            pl.BlockSpec(memory_space=pltpu.SMEM),
        ],
        out_specs=pl.BlockSpec((3, br, 128), lambda i: (0, i, 0)),
        out_shape=jax.ShapeDtypeStruct((3, m, 128), jnp.float32),
    )(dt, at, W1, b1, W2, b2)


def _sc_scatter(streams):
    mesh = plsc.VectorSubcoreMesh(core_axis_name="c", subcore_axis_name="s")
    zeros16 = None

    def body(*refs):
        stream_refs = refs[:4 * len(streams)]
        out_ref = refs[4 * len(streams)]
        idxb, yb, acc = refs[4 * len(streams) + 1:]
        cid = lax.axis_index("c")
        sid = lax.axis_index("s")
        w = sid * 2 + cid
        z16 = jnp.zeros((16,), jnp.float32)
        for c in range(3):
            def zero_body(i, _):
                acc[pl.ds(i * 16, 16)] = z16
                return 0
            lax.fori_loop(0, _NPAD // 16, zero_body, 0, unroll=8)
            for si in range(len(streams)):
                yc_ref = stream_refs[4 * si + c]
                dst_ref = stream_refs[4 * si + 3]
                ep = streams[si][3].shape[0]
                per = ep // 32
                tb = w * per
                nwin = -(-per // _WSC)
                for wi in range(nwin):
                    off = wi * _WSC
                    wlen = min(_WSC, per - off)
                    assert wlen % 16 == 0, (per, wlen)
                    pltpu.sync_copy(dst_ref.at[pl.ds(tb + off, wlen)],
                                    idxb.at[pl.ds(0, wlen)])
                    pltpu.sync_copy(yc_ref.at[pl.ds(tb + off, wlen)],
                                    yb.at[pl.ds(0, wlen)])

                    def add_body(i, _):
                        plsc.addupdate_scatter(
                            acc, [idxb[pl.ds(i * 16, 16)]],
                            yb[pl.ds(i * 16, 16)])
                        return 0

                    lax.fori_loop(0, wlen // 16, add_body, 0, unroll=8)
            pltpu.sync_copy(acc,
                            out_ref.at[pl.ds((w * 3 + c) * _NPAD, _NPAD)])

    args = []
    for st in streams:
        args += list(st)
    return pl.kernel(
        body,
        out_type=jax.ShapeDtypeStruct((32 * 3 * _NPAD,), jnp.float32),
        mesh=mesh,
        compiler_params=pltpu.CompilerParams(needs_layout_passes=False),
        scratch_types=[
            pltpu.VMEM((_WSC,), jnp.int32),
            pltpu.VMEM((_WSC,), jnp.float32),
            pltpu.VMEM((_NPAD,), jnp.float32),
        ],
    )(*args)


def _combine_body(t_ref, o_ref):
    for c in range(3):
        acc = t_ref[c]
        for wk in range(1, 32):
            acc = acc + t_ref[3 * wk + c]
        o_ref[c] = acc


def _combine(partials):
    m = _NPAD // 128
    t = partials.reshape(96, m, 128)
    bm = 56
    return pl.pallas_call(
        _combine_body,
        grid=(m // bm,),
        in_specs=[pl.BlockSpec((96, bm, 128), lambda i: (0, i, 0))],
        out_specs=pl.BlockSpec((3, bm, 128), lambda i: (0, i, 0)),
        out_shape=jax.ShapeDtypeStruct((3, m, 128), jnp.float32),
    )(t)


def _term(xt, edges, attr, W1, b1, W2, b2):
    e = attr.shape[0]
    ep = -(-e // 1024) * 1024
    m = ep // 128
    br = next(c for c in (64, 56, 32, 16, 8) if m % c == 0)
    if edges.shape[0] == 2:
        d = jnp.take(xt, edges[0], axis=1) - jnp.take(xt, edges[1], axis=1)
        dst = edges[1]
    else:
        xj = jnp.take(xt, edges[0], axis=1)
        xk = jnp.take(xt, edges[1], axis=1)
        xi = jnp.take(xt, edges[2], axis=1)
        d = jnp.concatenate((xk - xj, xi - xk), axis=0)
        dst = edges[2]
    a = attr.T
    d = jnp.pad(d, ((0, 0), (0, ep - e)))
    a = jnp.pad(a, ((0, 0), (0, ep - e)))
    yt = _edge_mlp_planes(d.reshape(-1, m, 128), a.reshape(3, m, 128),
                          W1, b1, W2, b2, br)
    return yt.reshape(3, ep), jnp.pad(dst, (0, ep - e))


def kernel(x, edge_2body, edge_3body, edge_2bodySelf, edge_1body,
           edge_attr_2body, edge_attr_3body, edge_attr_2bodySelf, edge_attr_1body,
           W1_2b, b1_2b, W2_2b, b2_2b,
           W1_3b, b1_3b, W2_3b, b2_3b,
           W1_s, b1_s, W2_s, b2_s):
    n = x.shape[0]
    xt = x.T
    y2, d2 = _term(xt, edge_2body, edge_attr_2body, W1_2b, b1_2b, W2_2b, b2_2b)
    y3, d3 = _term(xt, edge_3body, edge_attr_3body, W1_3b, b1_3b, W2_3b, b2_3b)
    ys, ds = _term(xt, edge_2bodySelf, edge_attr_2bodySelf, W1_s, b1_s, W2_s, b2_s)
    parts = _sc_scatter([(y2[0], y2[1], y2[2], d2),
                         (y3[0], y3[1], y3[2], d3),
                         (ys[0], ys[1], ys[2], ds)])
    v = _combine(parts).reshape(3, _NPAD)
    return v[:, :n].T


# speedup vs baseline: 2.8127x; 1.9090x over previous
import functools

import jax
import jax.numpy as jnp
from jax import lax
from jax.experimental import pallas as pl
from jax.experimental.pallas import tpu as pltpu
from jax.experimental.pallas import tpu_sc as plsc

_HID = 32
_NPAD = 50176
_WSC = 12544


def _mlp_plane_body(d_ref, a_ref, W1_ref, b1_ref, W2_ref, b2_ref, y_ref, *, din):
    o = [None] * 9
    for k in range(_HID):
        h = d_ref[0] * W1_ref[0, k]
        for j in range(1, din):
            h = h + d_ref[j] * W1_ref[j, k]
        h = jnp.maximum(h + b1_ref[k], 0.0)
        for r in range(9):
            if k == 0:
                o[r] = h * W2_ref[k, r] + b2_ref[r]
            else:
                o[r] = o[r] + h * W2_ref[k, r]
    a0, a1, a2 = a_ref[0], a_ref[1], a_ref[2]
    for i in range(3):
        y_ref[i] = o[3 * i] * a0 + o[3 * i + 1] * a1 + o[3 * i + 2] * a2


def _edge_mlp_planes(dt, at, W1, b1, W2, b2, br):
    din, m, _ = dt.shape
    assert m % br == 0, (m, br)
    grid = m // br
    return pl.pallas_call(
        functools.partial(_mlp_plane_body, din=din),
        grid=(grid,),
        in_specs=[
            pl.BlockSpec((din, br, 128), lambda i: (0, i, 0)),
            pl.BlockSpec((3, br, 128), lambda i: (0, i, 0)),
            pl.BlockSpec(memory_space=pltpu.SMEM),
            pl.BlockSpec(memory_space=pltpu.SMEM),
            pl.BlockSpec(memory_space=pltpu.SMEM),
            pl.BlockSpec(memory_space=pltpu.SMEM),
        ],
        out_specs=pl.BlockSpec((3, br, 128), lambda i: (0, i, 0)),
        out_shape=jax.ShapeDtypeStruct((3, m, 128), jnp.float32),
    )(dt, at, W1, b1, W2, b2)


_WG = 6272


def _sc_gather(xt_flat, e2s, e2d, a2f, e3j, e3k, e3i, a3f, ess, esd, asf):
    ep2 = e2s.shape[0]
    ep3 = e3j.shape[0]
    eps = ess.shape[0]
    per2, per3, pers = ep2 // 32, ep3 // 32, eps // 32
    assert per2 % _WG == 0 and per3 % _WG == 0 and pers <= _WG
    assert pers % 16 == 0
    mesh = plsc.VectorSubcoreMesh(core_axis_name="c", subcore_axis_name="s")

    def body(xt, e2s_r, e2d_r, a2f_r, e3j_r, e3k_r, e3i_r, a3f_r,
             ess_r, esd_r, asf_r,
             d2o, a2o, d3o, a3o, dso, aso,
             xplane, ib1, ib2, ib3, db1, db2, abuf, ap0, ap1, ap2):
        cid = lax.axis_index("c")
        sid = lax.axis_index("s")
        w = sid * 2 + cid
        iota3 = lax.iota(jnp.int32, 16) * 3
        aps = (ap0, ap1, ap2)

        def two_body_win(base, wlen, src_r, dst_r, out_r, out_off):
            pltpu.sync_copy(src_r.at[pl.ds(base, wlen)],
                            ib1.at[pl.ds(0, wlen)])
            pltpu.sync_copy(dst_r.at[pl.ds(base, wlen)],
                            ib2.at[pl.ds(0, wlen)])

            def g(i, _):
                sl = pl.ds(i * 16, 16)
                vs = plsc.load_gather(xplane, [ib1[sl]])
                vt = plsc.load_gather(xplane, [ib2[sl]])
                db1[sl] = vs - vt
                return 0

            lax.fori_loop(0, wlen // 16, g, 0, unroll=4)
            pltpu.sync_copy(db1.at[pl.ds(0, wlen)],
                            out_r.at[pl.ds(out_off + base, wlen)])

        def three_body_win(base, wlen, c):
            pltpu.sync_copy(e3j_r.at[pl.ds(base, wlen)],
                            ib1.at[pl.ds(0, wlen)])
            pltpu.sync_copy(e3k_r.at[pl.ds(base, wlen)],
                            ib2.at[pl.ds(0, wlen)])
            pltpu.sync_copy(e3i_r.at[pl.ds(base, wlen)],
                            ib3.at[pl.ds(0, wlen)])

            def g(i, _):
                sl = pl.ds(i * 16, 16)
                vj = plsc.load_gather(xplane, [ib1[sl]])
                vk = plsc.load_gather(xplane, [ib2[sl]])
                vi = plsc.load_gather(xplane, [ib3[sl]])
                db1[sl] = vk - vj
                db2[sl] = vi - vk
                return 0

            lax.fori_loop(0, wlen // 16, g, 0, unroll=4)
            pltpu.sync_copy(db1.at[pl.ds(0, wlen)],
                            d3o.at[pl.ds(c * ep3 + base, wlen)])
            pltpu.sync_copy(db2.at[pl.ds(0, wlen)],
                            d3o.at[pl.ds((3 + c) * ep3 + base, wlen)])

        for c in range(3):
            pltpu.sync_copy(xt.at[pl.ds(c * _NPAD, _NPAD)], xplane)

            def w2(wi, _, c=c):
                two_body_win(w * per2 + wi * _WG, _WG, e2s_r, e2d_r,
                             d2o, c * ep2)
                return 0

            lax.fori_loop(0, per2 // _WG, w2, 0)

            def w3(wi, _, c=c):
                three_body_win(w * per3 + wi * _WG, _WG, c)
                return 0

            lax.fori_loop(0, per3 // _WG, w3, 0)
            two_body_win(w * pers, pers, ess_r, esd_r, dso, c * eps)

        def attr_pass(a_r, out_r, ep, per, nwin, wlen):
            def aw(wi, _):
                base = w * per + wi * wlen

                def ag(i, _):
                    p0 = i * 48 + iota3
                    for c in range(3):
                        aps[c][pl.ds(i * 16, 16)] = plsc.load_gather(
                            abuf, [p0 + c])
                    return 0

                pltpu.sync_copy(a_r.at[pl.ds(3 * base, 3 * wlen)],
                                abuf.at[pl.ds(0, 3 * wlen)])
                lax.fori_loop(0, wlen // 16, ag, 0, unroll=4)
                for c in range(3):
                    pltpu.sync_copy(aps[c].at[pl.ds(0, wlen)],
                                    out_r.at[pl.ds(c * ep + base, wlen)])
                return 0

            lax.fori_loop(0, nwin, aw, 0)

        attr_pass(a2f_r, a2o, ep2, per2, per2 // _WG, _WG)
        attr_pass(a3f_r, a3o, ep3, per3, per3 // _WG, _WG)
        attr_pass(asf_r, aso, eps, pers, 1, pers)

    f32 = jnp.float32
    return pl.kernel(
        body,
        out_type=(
            jax.ShapeDtypeStruct((3 * ep2,), f32),
            jax.ShapeDtypeStruct((3 * ep2,), f32),
            jax.ShapeDtypeStruct((6 * ep3,), f32),
            jax.ShapeDtypeStruct((3 * ep3,), f32),
            jax.ShapeDtypeStruct((3 * eps,), f32),
            jax.ShapeDtypeStruct((3 * eps,), f32),
        ),
        mesh=mesh,
        compiler_params=pltpu.CompilerParams(needs_layout_passes=False),
        scratch_types=[
            pltpu.VMEM((_NPAD,), f32),
            pltpu.VMEM((_WG,), jnp.int32),
            pltpu.VMEM((_WG,), jnp.int32),
            pltpu.VMEM((_WG,), jnp.int32),
            pltpu.VMEM((_WG,), f32),
            pltpu.VMEM((_WG,), f32),
            pltpu.VMEM((3 * _WG,), f32),
            pltpu.VMEM((_WG,), f32),
            pltpu.VMEM((_WG,), f32),
            pltpu.VMEM((_WG,), f32),
        ],
    )(xt_flat, e2s, e2d, a2f, e3j, e3k, e3i, a3f, ess, esd, asf)


def _sc_scatter(streams):
    mesh = plsc.VectorSubcoreMesh(core_axis_name="c", subcore_axis_name="s")
    zeros16 = None

    def body(*refs):
        stream_refs = refs[:4 * len(streams)]
        out_ref = refs[4 * len(streams)]
        idxb, yb, acc = refs[4 * len(streams) + 1:]
        cid = lax.axis_index("c")
        sid = lax.axis_index("s")
        w = sid * 2 + cid
        z16 = jnp.zeros((16,), jnp.float32)
        for c in range(3):
            def zero_body(i, _):
                acc[pl.ds(i * 16, 16)] = z16
                return 0
            lax.fori_loop(0, _NPAD // 16, zero_body, 0, unroll=8)
            for si in range(len(streams)):
                yc_ref = stream_refs[4 * si + c]
                dst_ref = stream_refs[4 * si + 3]
                ep = streams[si][3].shape[0]
                per = ep // 32
                tb = w * per
                nwin = -(-per // _WSC)
                for wi in range(nwin):
                    off = wi * _WSC
                    wlen = min(_WSC, per - off)
                    assert wlen % 16 == 0, (per, wlen)
                    pltpu.sync_copy(dst_ref.at[pl.ds(tb + off, wlen)],
                                    idxb.at[pl.ds(0, wlen)])
                    pltpu.sync_copy(yc_ref.at[pl.ds(tb + off, wlen)],
                                    yb.at[pl.ds(0, wlen)])

                    def add_body(i, _):
                        plsc.addupdate_scatter(
                            acc, [idxb[pl.ds(i * 16, 16)]],
                            yb[pl.ds(i * 16, 16)])
                        return 0

                    lax.fori_loop(0, wlen // 16, add_body, 0, unroll=8)
            pltpu.sync_copy(acc,
                            out_ref.at[pl.ds((w * 3 + c) * _NPAD, _NPAD)])

    args = []
    for st in streams:
        args += list(st)
    return pl.kernel(
        body,
        out_type=jax.ShapeDtypeStruct((32 * 3 * _NPAD,), jnp.float32),
        mesh=mesh,
        compiler_params=pltpu.CompilerParams(needs_layout_passes=False),
        scratch_types=[
            pltpu.VMEM((_WSC,), jnp.int32),
            pltpu.VMEM((_WSC,), jnp.float32),
            pltpu.VMEM((_NPAD,), jnp.float32),
        ],
    )(*args)


def _combine_body(t_ref, o_ref):
    for c in range(3):
        acc = t_ref[c]
        for wk in range(1, 32):
            acc = acc + t_ref[3 * wk + c]
        o_ref[c] = acc


def _combine(partials):
    m = _NPAD // 128
    t = partials.reshape(96, m, 128)
    bm = 56
    return pl.pallas_call(
        _combine_body,
        grid=(m // bm,),
        in_specs=[pl.BlockSpec((96, bm, 128), lambda i: (0, i, 0))],
        out_specs=pl.BlockSpec((3, bm, 128), lambda i: (0, i, 0)),
        out_shape=jax.ShapeDtypeStruct((3, m, 128), jnp.float32),
    )(t)


def kernel(x, edge_2body, edge_3body, edge_2bodySelf, edge_1body,
           edge_attr_2body, edge_attr_3body, edge_attr_2bodySelf, edge_attr_1body,
           W1_2b, b1_2b, W2_2b, b2_2b,
           W1_3b, b1_3b, W2_3b, b2_3b,
           W1_s, b1_s, W2_s, b2_s):
    n = x.shape[0]
    e2 = edge_attr_2body.shape[0]
    e3 = edge_attr_3body.shape[0]
    es = edge_attr_2bodySelf.shape[0]
    ep2 = -(-e2 // (32 * _WG)) * (32 * _WG)
    ep3 = -(-e3 // (32 * _WG)) * (32 * _WG)
    eps = -(-es // 1024) * 1024

    def padi(a, ep):
        return jnp.pad(a, (0, ep - a.shape[0]))

    xtp = jnp.pad(x.T, ((0, 0), (0, _NPAD - n))).reshape(-1)
    a2f = jnp.pad(edge_attr_2body, ((0, ep2 - e2), (0, 0))).reshape(-1)
    a3f = jnp.pad(edge_attr_3body, ((0, ep3 - e3), (0, 0))).reshape(-1)
    asf = jnp.pad(edge_attr_2bodySelf, ((0, eps - es), (0, 0))).reshape(-1)
    e2d = padi(edge_2body[1], ep2)
    e3i = padi(edge_3body[2], ep3)
    esd = padi(edge_2bodySelf[1], eps)
    d2f, a2t, d3f, a3t, dsf, ast = _sc_gather(
        xtp, padi(edge_2body[0], ep2), e2d, a2f,
        padi(edge_3body[0], ep3), padi(edge_3body[1], ep3), e3i, a3f,
        padi(edge_2bodySelf[0], eps), esd, asf)
    m2, m3, ms = ep2 // 128, ep3 // 128, eps // 128
    y2 = _edge_mlp_planes(d2f.reshape(3, m2, 128), a2t.reshape(3, m2, 128),
                          W1_2b, b1_2b, W2_2b, b2_2b,
                          next(c for c in (64, 56, 32, 16, 8) if m2 % c == 0))
    y3 = _edge_mlp_planes(d3f.reshape(6, m3, 128), a3t.reshape(3, m3, 128),
                          W1_3b, b1_3b, W2_3b, b2_3b,
                          next(c for c in (64, 56, 32, 16, 8) if m3 % c == 0))
    ys = _edge_mlp_planes(dsf.reshape(3, ms, 128), ast.reshape(3, ms, 128),
                          W1_s, b1_s, W2_s, b2_s,
                          next(c for c in (64, 56, 32, 16, 8) if ms % c == 0))
    y2 = y2.reshape(3, ep2)
    y3 = y3.reshape(3, ep3)
    ys = ys.reshape(3, eps)
    parts = _sc_scatter([(y2[0], y2[1], y2[2], e2d),
                         (y3[0], y3[1], y3[2], e3i),
                         (ys[0], ys[1], ys[2], esd)])
    v = _combine(parts).reshape(3, _NPAD)
    return v[:, :n].T


# speedup vs baseline: 3.1608x; 1.1238x over previous
import functools

import jax
import jax.numpy as jnp
from jax import lax
from jax.experimental import pallas as pl
from jax.experimental.pallas import tpu as pltpu
from jax.experimental.pallas import tpu_sc as plsc

_HID = 32
_NPAD = 50176
_WG = 8000
_WSC = 10000

_SC_PARAMS = pltpu.CompilerParams(needs_layout_passes=False)
_MESH = dict(core_axis_name="c", subcore_axis_name="s")


def _wins(per, wmax):
    out, off = [], 0
    while off < per:
        wl = min(wmax, per - off)
        assert wl % 16 == 0, (per, wmax)
        out.append((off, wl))
        off += wl
    return out


def _self_split(e):
    com = (e // 32) // 16 * 16
    rem = e - 32 * com
    assert rem % 16 == 0
    return com, rem


def _sc_gather_x(xt_flat, e2s, e2d, e3j, e3k, e3i, ess, esd, s2, s3):
    n3 = xt_flat.shape[0] // 3
    e2, e3, es = e2s.shape[0], e3j.shape[0], ess.shape[0]
    assert e2 % 32 == 0 and e3 % 32 == 0
    w2, w3 = _wins(e2 // 32, _WG), _wins(e3 // 32, _WG)
    scom, srem = _self_split(es)
    mesh = plsc.VectorSubcoreMesh(**_MESH)

    def body(xt, e2s_r, e2d_r, e3j_r, e3k_r, e3i_r, ess_r, esd_r,
             d2o, d3o, dso, xplane, ib1, ib2, ib3, db1, db2):
        w = lax.axis_index("s") * 2 + lax.axis_index("c")

        def diff_win(base, wlen, src_r, dst_r, out_r, out_off):
            pltpu.sync_copy(src_r.at[pl.ds(base, wlen)],
                            ib1.at[pl.ds(0, wlen)])
            pltpu.sync_copy(dst_r.at[pl.ds(base, wlen)],
                            ib2.at[pl.ds(0, wlen)])

            def g(i, _):
                sl = pl.ds(i * 16, 16)
                db1[sl] = (plsc.load_gather(xplane, [ib1[sl]]) -
                           plsc.load_gather(xplane, [ib2[sl]]))
                return 0

            lax.fori_loop(0, wlen // 16, g, 0, unroll=4)
            pltpu.sync_copy(db1.at[pl.ds(0, wlen)],
                            out_r.at[pl.ds(out_off + base, wlen)])

        def tri_win(base, wlen, c):
            pltpu.sync_copy(e3j_r.at[pl.ds(base, wlen)],
                            ib1.at[pl.ds(0, wlen)])
            pltpu.sync_copy(e3k_r.at[pl.ds(base, wlen)],
                            ib2.at[pl.ds(0, wlen)])
            pltpu.sync_copy(e3i_r.at[pl.ds(base, wlen)],
                            ib3.at[pl.ds(0, wlen)])

            def g(i, _):
                sl = pl.ds(i * 16, 16)
                vj = plsc.load_gather(xplane, [ib1[sl]])
                vk = plsc.load_gather(xplane, [ib2[sl]])
                vi = plsc.load_gather(xplane, [ib3[sl]])
                db1[sl] = vk - vj
                db2[sl] = vi - vk
                return 0

            lax.fori_loop(0, wlen // 16, g, 0, unroll=4)
            pltpu.sync_copy(db1.at[pl.ds(0, wlen)],
                            d3o.at[pl.ds(c * s3 + base, wlen)])
            pltpu.sync_copy(db2.at[pl.ds(0, wlen)],
                            d3o.at[pl.ds((3 + c) * s3 + base, wlen)])

        for c in range(3):
            pltpu.sync_copy(xt.at[pl.ds(c * n3, n3)],
                            xplane.at[pl.ds(0, n3)])
            for off, wl in w2:
                diff_win(w * (e2 // 32) + off, wl, e2s_r, e2d_r, d2o, c * s2)
            for off, wl in w3:
                tri_win(w * (e3 // 32) + off, wl, c)
            diff_win(w * scom, scom, ess_r, esd_r, dso, c * _NPAD)

            @pl.when(w == 0)
            def _(c=c):
                diff_win(32 * scom, srem, ess_r, esd_r, dso, c * _NPAD)

    f32 = jnp.float32
    return pl.kernel(
        body,
        out_type=(
            jax.ShapeDtypeStruct((3 * s2,), f32),
            jax.ShapeDtypeStruct((6 * s3,), f32),
            jax.ShapeDtypeStruct((3 * _NPAD,), f32),
        ),
        mesh=mesh,
        compiler_params=_SC_PARAMS,
        scratch_types=[
            pltpu.VMEM((_NPAD,), f32),
            pltpu.VMEM((_WG,), jnp.int32),
            pltpu.VMEM((_WG,), jnp.int32),
            pltpu.VMEM((_WG,), jnp.int32),
            pltpu.VMEM((_WG,), f32),
            pltpu.VMEM((_WG,), f32),
        ],
    )(xt_flat, e2s, e2d, e3j, e3k, e3i, ess, esd)


def _sc_attr_t(a2f, a3f, asf, s2, s3):
    e2, e3, es = a2f.shape[0] // 3, a3f.shape[0] // 3, asf.shape[0] // 3
    w2, w3 = _wins(e2 // 32, _WG), _wins(e3 // 32, _WG)
    scom, srem = _self_split(es)
    mesh = plsc.VectorSubcoreMesh(**_MESH)

    def body(a2_r, a3_r, as_r, a2o, a3o, aso, abuf, ap0, ap1, ap2):
        w = lax.axis_index("s") * 2 + lax.axis_index("c")
        iota3 = lax.iota(jnp.int32, 16) * 3
        aps = (ap0, ap1, ap2)

        def attr_win(base, wlen, a_r, out_r, stride):
            pltpu.sync_copy(a_r.at[pl.ds(3 * base, 3 * wlen)],
                            abuf.at[pl.ds(0, 3 * wlen)])

            def g(i, _):
                p0 = i * 48 + iota3
                for c in range(3):
                    aps[c][pl.ds(i * 16, 16)] = plsc.load_gather(
                        abuf, [p0 + c])
                return 0

            lax.fori_loop(0, wlen // 16, g, 0, unroll=4)
            for c in range(3):
                pltpu.sync_copy(aps[c].at[pl.ds(0, wlen)],
                                out_r.at[pl.ds(c * stride + base, wlen)])

        for off, wl in w2:
            attr_win(w * (e2 // 32) + off, wl, a2_r, a2o, s2)
        for off, wl in w3:
            attr_win(w * (e3 // 32) + off, wl, a3_r, a3o, s3)
        attr_win(w * scom, scom, as_r, aso, _NPAD)

        @pl.when(w == 0)
        def _():
            attr_win(32 * scom, srem, as_r, aso, _NPAD)

    f32 = jnp.float32
    return pl.kernel(
        body,
        out_type=(
            jax.ShapeDtypeStruct((3 * s2,), f32),
            jax.ShapeDtypeStruct((3 * s3,), f32),
            jax.ShapeDtypeStruct((3 * _NPAD,), f32),
        ),
        mesh=mesh,
        compiler_params=_SC_PARAMS,
        scratch_types=[
            pltpu.VMEM((3 * _WG,), f32),
            pltpu.VMEM((_WG,), f32),
            pltpu.VMEM((_WG,), f32),
            pltpu.VMEM((_WG,), f32),
        ],
    )(a2f, a3f, asf)


def _sc_scatter(streams):
    mesh = plsc.VectorSubcoreMesh(**_MESH)

    def body(*refs):
        stream_refs = refs[:2 * len(streams)]
        out_ref = refs[2 * len(streams)]
        idxb, yb, acc = refs[2 * len(streams) + 1:]
        w = lax.axis_index("s") * 2 + lax.axis_index("c")
        z16 = jnp.zeros((16,), jnp.float32)

        def add_win(base, wlen, y_r, dst_r, c, stride):
            pltpu.sync_copy(dst_r.at[pl.ds(base, wlen)],
                            idxb.at[pl.ds(0, wlen)])
            pltpu.sync_copy(y_r.at[pl.ds(c * stride + base, wlen)],
                            yb.at[pl.ds(0, wlen)])

            def g(i, _):
                sl = pl.ds(i * 16, 16)
                plsc.addupdate_scatter(acc, [idxb[sl]], yb[sl])
                return 0

            lax.fori_loop(0, wlen // 16, g, 0, unroll=8)

        for c in range(3):
            def zero_body(i, _):
                acc[pl.ds(i * 16, 16)] = z16
                return 0

            lax.fori_loop(0, _NPAD // 16, zero_body, 0, unroll=8)
            for si in range(len(streams)):
                y_r = stream_refs[2 * si]
                dst_r = stream_refs[2 * si + 1]
                e = streams[si][1].shape[0]
                stride = streams[si][2]
                if e % 32 == 0 and (e // 32) % 16 == 0:
                    for off, wl in _wins(e // 32, _WSC):
                        add_win(w * (e // 32) + off, wl, y_r, dst_r, c,
                                stride)
                else:
                    scom, srem = _self_split(e)
                    add_win(w * scom, scom, y_r, dst_r, c, stride)

                    @pl.when(w == 0)
                    def _(y_r=y_r, dst_r=dst_r, c=c, stride=stride,
                          scom=scom, srem=srem):
                        add_win(32 * scom, srem, y_r, dst_r, c, stride)
            pltpu.sync_copy(acc,
                            out_ref.at[pl.ds((w * 3 + c) * _NPAD, _NPAD)])

    args = []
    for y, dst, stride in streams:
        args += [y, dst]
    return pl.kernel(
        body,
        out_type=jax.ShapeDtypeStruct((32 * 3 * _NPAD,), jnp.float32),
        mesh=mesh,
        compiler_params=_SC_PARAMS,
        scratch_types=[
            pltpu.VMEM((_WSC,), jnp.int32),
            pltpu.VMEM((_WSC,), jnp.float32),
            pltpu.VMEM((_NPAD,), jnp.float32),
        ],
    )(*args)


def _mlp_plane_body(d_ref, a_ref, W1_ref, b1_ref, W2_ref, b2_ref, y_ref, *, din):
    o = [None] * 9
    for k in range(_HID):
        h = d_ref[0] * W1_ref[0, k]
        for j in range(1, din):
            h = h + d_ref[j] * W1_ref[j, k]
        h = jnp.maximum(h + b1_ref[k], 0.0)
        for r in range(9):
            if k == 0:
                o[r] = h * W2_ref[k, r] + b2_ref[r]
            else:
                o[r] = o[r] + h * W2_ref[k, r]
    a0, a1, a2 = a_ref[0], a_ref[1], a_ref[2]
    for i in range(3):
        y_ref[i] = o[3 * i] * a0 + o[3 * i + 1] * a1 + o[3 * i + 2] * a2


def _edge_mlp_planes(dt, at, W1, b1, W2, b2):
    din, m, _ = dt.shape
    br = next(c for c in (64, 56, 50, 32, 16, 8, 4) if m % c == 0)
    grid = m // br
    return pl.pallas_call(
        functools.partial(_mlp_plane_body, din=din),
        grid=(grid,),
        in_specs=[
            pl.BlockSpec((din, br, 128), lambda i: (0, i, 0)),
            pl.BlockSpec((3, br, 128), lambda i: (0, i, 0)),
            pl.BlockSpec(memory_space=pltpu.SMEM),
            pl.BlockSpec(memory_space=pltpu.SMEM),
            pl.BlockSpec(memory_space=pltpu.SMEM),
            pl.BlockSpec(memory_space=pltpu.SMEM),
        ],
        out_specs=pl.BlockSpec((3, br, 128), lambda i: (0, i, 0)),
        out_shape=jax.ShapeDtypeStruct((3, m, 128), jnp.float32),
    )(dt, at, W1, b1, W2, b2)


def _combine_body(t_ref, o_ref):
    for c in range(3):
        acc = t_ref[c]
        for wk in range(1, 32):
            acc = acc + t_ref[3 * wk + c]
        o_ref[c] = acc


def _combine(partials):
    m = _NPAD // 128
    t = partials.reshape(96, m, 128)
    bm = 56
    return pl.pallas_call(
        _combine_body,
        grid=(m // bm,),
        in_specs=[pl.BlockSpec((96, bm, 128), lambda i: (0, i, 0))],
        out_specs=pl.BlockSpec((3, bm, 128), lambda i: (0, i, 0)),
        out_shape=jax.ShapeDtypeStruct((3, m, 128), jnp.float32),
    )(t)


def kernel(x, edge_2body, edge_3body, edge_2bodySelf, edge_1body,
           edge_attr_2body, edge_attr_3body, edge_attr_2bodySelf, edge_attr_1body,
           W1_2b, b1_2b, W2_2b, b2_2b,
           W1_3b, b1_3b, W2_3b, b2_3b,
           W1_s, b1_s, W2_s, b2_s):
    n = x.shape[0]
    e2 = edge_attr_2body.shape[0]
    e3 = edge_attr_3body.shape[0]
    s2 = -(-e2 // 8192) * 8192
    s3 = -(-e3 // 8192) * 8192

    xtf = x.T.reshape(-1)
    d2f, d3f, dsf = _sc_gather_x(
        xtf, edge_2body[0], edge_2body[1],
        edge_3body[0], edge_3body[1], edge_3body[2],
        edge_2bodySelf[0], edge_2bodySelf[1], s2, s3)
    a2t, a3t, ast = _sc_attr_t(
        edge_attr_2body.reshape(-1), edge_attr_3body.reshape(-1),
        edge_attr_2bodySelf.reshape(-1), s2, s3)
    y2 = _edge_mlp_planes(d2f.reshape(3, s2 // 128, 128),
                          a2t.reshape(3, s2 // 128, 128),
                          W1_2b, b1_2b, W2_2b, b2_2b)
    y3 = _edge_mlp_planes(d3f.reshape(6, s3 // 128, 128),
                          a3t.reshape(3, s3 // 128, 128),
                          W1_3b, b1_3b, W2_3b, b2_3b)
    ys = _edge_mlp_planes(dsf.reshape(3, _NPAD // 128, 128),
                          ast.reshape(3, _NPAD // 128, 128),
                          W1_s, b1_s, W2_s, b2_s)
    parts = _sc_scatter([
        (y2.reshape(-1), edge_2body[1], s2),
        (y3.reshape(-1), edge_3body[2], s3),
        (ys.reshape(-1), edge_2bodySelf[1], _NPAD),
    ])
    v = _combine(parts).reshape(3, _NPAD)
    return v[:, :n].T


# speedup vs baseline: 27.1132x; 8.5781x over previous
import functools

import jax
import jax.numpy as jnp
from jax import lax
from jax.experimental import pallas as pl
from jax.experimental.pallas import tpu as pltpu
from jax.experimental.pallas import tpu_sc as plsc

_HID = 32
_NPAD = 50176
_WG = 8000
_WSC = 10000

_SC_PARAMS = pltpu.CompilerParams(needs_layout_passes=False)
_MESH = dict(core_axis_name="c", subcore_axis_name="s")


def _wins(per, wmax):
    out, off = [], 0
    while off < per:
        wl = min(wmax, per - off)
        assert wl % 16 == 0, (per, wmax)
        out.append((off, wl))
        off += wl
    return out


def _self_split(e):
    com = (e // 32) // 16 * 16
    rem = e - 32 * com
    assert rem % 16 == 0
    return com, rem


def _sc_gather_x(xt_flat, e2s, e2d, e3j, e3k, e3i, ess, esd, s2, s3):
    n3 = xt_flat.shape[0] // 3
    e2, e3, es = e2s.shape[0], e3j.shape[0], ess.shape[0]
    assert e2 % 32 == 0 and e3 % 32 == 0
    w2, w3 = _wins(e2 // 32, _WG), _wins(e3 // 32, _WG)
    scom, srem = _self_split(es)
    mesh = plsc.VectorSubcoreMesh(**_MESH)

    def body(xt, e2s_r, e2d_r, e3j_r, e3k_r, e3i_r, ess_r, esd_r,
             d2o, d3o, dso, xplane, ib1, ib2, ib3, db1, db2):
        w = lax.axis_index("s") * 2 + lax.axis_index("c")

        def diff_win(base, wlen, src_r, dst_r, out_r, out_off):
            pltpu.sync_copy(src_r.at[pl.ds(base, wlen)],
                            ib1.at[pl.ds(0, wlen)])
            pltpu.sync_copy(dst_r.at[pl.ds(base, wlen)],
                            ib2.at[pl.ds(0, wlen)])

            def g(i, _):
                sl = pl.ds(i * 16, 16)
                db1[sl] = (plsc.load_gather(xplane, [ib1[sl]]) -
                           plsc.load_gather(xplane, [ib2[sl]]))
                return 0

            lax.fori_loop(0, wlen // 16, g, 0, unroll=4)
            pltpu.sync_copy(db1.at[pl.ds(0, wlen)],
                            out_r.at[pl.ds(out_off + base, wlen)])

        def tri_win(base, wlen, c):
            pltpu.sync_copy(e3j_r.at[pl.ds(base, wlen)],
                            ib1.at[pl.ds(0, wlen)])
            pltpu.sync_copy(e3k_r.at[pl.ds(base, wlen)],
                            ib2.at[pl.ds(0, wlen)])
            pltpu.sync_copy(e3i_r.at[pl.ds(base, wlen)],
                            ib3.at[pl.ds(0, wlen)])

            def g(i, _):
                sl = pl.ds(i * 16, 16)
                vj = plsc.load_gather(xplane, [ib1[sl]])
                vk = plsc.load_gather(xplane, [ib2[sl]])
                vi = plsc.load_gather(xplane, [ib3[sl]])
                db1[sl] = vk - vj
                db2[sl] = vi - vk
                return 0

            lax.fori_loop(0, wlen // 16, g, 0, unroll=4)
            pltpu.sync_copy(db1.at[pl.ds(0, wlen)],
                            d3o.at[pl.ds(c * s3 + base, wlen)])
            pltpu.sync_copy(db2.at[pl.ds(0, wlen)],
                            d3o.at[pl.ds((3 + c) * s3 + base, wlen)])

        for c in range(3):
            pltpu.sync_copy(xt.at[pl.ds(c * n3, n3)],
                            xplane.at[pl.ds(0, n3)])
            for off, wl in w2:
                diff_win(w * (e2 // 32) + off, wl, e2s_r, e2d_r, d2o, c * s2)
            for off, wl in w3:
                tri_win(w * (e3 // 32) + off, wl, c)
            diff_win(w * scom, scom, ess_r, esd_r, dso, c * _NPAD)

            @pl.when(w == 0)
            def _(c=c):
                diff_win(32 * scom, srem, ess_r, esd_r, dso, c * _NPAD)

    f32 = jnp.float32
    return pl.kernel(
        body,
        out_type=(
            jax.ShapeDtypeStruct((3 * s2,), f32),
            jax.ShapeDtypeStruct((6 * s3,), f32),
            jax.ShapeDtypeStruct((3 * _NPAD,), f32),
        ),
        mesh=mesh,
        compiler_params=_SC_PARAMS,
        scratch_types=[
            pltpu.VMEM((_NPAD,), f32),
            pltpu.VMEM((_WG,), jnp.int32),
            pltpu.VMEM((_WG,), jnp.int32),
            pltpu.VMEM((_WG,), jnp.int32),
            pltpu.VMEM((_WG,), f32),
            pltpu.VMEM((_WG,), f32),
        ],
    )(xt_flat, e2s, e2d, e3j, e3k, e3i, ess, esd)


def _sc_attr_t(a2f, a3f, asf, s2, s3):
    e2, e3, es = a2f.shape[0] // 3, a3f.shape[0] // 3, asf.shape[0] // 3
    w2, w3 = _wins(e2 // 32, _WG), _wins(e3 // 32, _WG)
    scom, srem = _self_split(es)
    mesh = plsc.VectorSubcoreMesh(**_MESH)

    def body(a2_r, a3_r, as_r, a2o, a3o, aso, abuf, ap0, ap1, ap2):
        w = lax.axis_index("s") * 2 + lax.axis_index("c")
        iota3 = lax.iota(jnp.int32, 16) * 3
        aps = (ap0, ap1, ap2)

        def attr_win(base, wlen, a_r, out_r, stride):
            pltpu.sync_copy(a_r.at[pl.ds(3 * base, 3 * wlen)],
                            abuf.at[pl.ds(0, 3 * wlen)])

            def g(i, _):
                p0 = i * 48 + iota3
                for c in range(3):
                    aps[c][pl.ds(i * 16, 16)] = plsc.load_gather(
                        abuf, [p0 + c])
                return 0

            lax.fori_loop(0, wlen // 16, g, 0, unroll=4)
            for c in range(3):
                pltpu.sync_copy(aps[c].at[pl.ds(0, wlen)],
                                out_r.at[pl.ds(c * stride + base, wlen)])

        for off, wl in w2:
            attr_win(w * (e2 // 32) + off, wl, a2_r, a2o, s2)
        for off, wl in w3:
            attr_win(w * (e3 // 32) + off, wl, a3_r, a3o, s3)
        attr_win(w * scom, scom, as_r, aso, _NPAD)

        @pl.when(w == 0)
        def _():
            attr_win(32 * scom, srem, as_r, aso, _NPAD)

    f32 = jnp.float32
    return pl.kernel(
        body,
        out_type=(
            jax.ShapeDtypeStruct((3 * s2,), f32),
            jax.ShapeDtypeStruct((3 * s3,), f32),
            jax.ShapeDtypeStruct((3 * _NPAD,), f32),
        ),
        mesh=mesh,
        compiler_params=_SC_PARAMS,
        scratch_types=[
            pltpu.VMEM((3 * _WG,), f32),
            pltpu.VMEM((_WG,), f32),
            pltpu.VMEM((_WG,), f32),
            pltpu.VMEM((_WG,), f32),
        ],
    )(a2f, a3f, asf)


def _sc_scatter(streams):
    mesh = plsc.VectorSubcoreMesh(**_MESH)

    def body(*refs):
        stream_refs = refs[:2 * len(streams)]
        out_ref = refs[2 * len(streams)]
        idxb, yb, acc = refs[2 * len(streams) + 1:]
        w = lax.axis_index("s") * 2 + lax.axis_index("c")
        z16 = jnp.zeros((16,), jnp.float32)

        def add_win(base, wlen, y_r, dst_r, c, stride):
            pltpu.sync_copy(dst_r.at[pl.ds(base, wlen)],
                            idxb.at[pl.ds(0, wlen)])
            pltpu.sync_copy(y_r.at[pl.ds(c * stride + base, wlen)],
                            yb.at[pl.ds(0, wlen)])

            def g(i, _):
                sl = pl.ds(i * 16, 16)
                plsc.addupdate_scatter(acc, [idxb[sl]], yb[sl])
                return 0

            lax.fori_loop(0, wlen // 16, g, 0, unroll=8)

        for c in range(3):
            def zero_body(i, _):
                acc[pl.ds(i * 16, 16)] = z16
                return 0

            lax.fori_loop(0, _NPAD // 16, zero_body, 0, unroll=8)
            for si in range(len(streams)):
                y_r = stream_refs[2 * si]
                dst_r = stream_refs[2 * si + 1]
                e = streams[si][1].shape[0]
                stride = streams[si][2]
                if e % 32 == 0 and (e // 32) % 16 == 0:
                    for off, wl in _wins(e // 32, _WSC):
                        add_win(w * (e // 32) + off, wl, y_r, dst_r, c,
                                stride)
                else:
                    scom, srem = _self_split(e)
                    add_win(w * scom, scom, y_r, dst_r, c, stride)

                    @pl.when(w == 0)
                    def _(y_r=y_r, dst_r=dst_r, c=c, stride=stride,
                          scom=scom, srem=srem):
                        add_win(32 * scom, srem, y_r, dst_r, c, stride)
            pltpu.sync_copy(acc,
                            out_ref.at[pl.ds((w * 3 + c) * _NPAD, _NPAD)])

    args = []
    for y, dst, stride in streams:
        args += [y, dst]
    return pl.kernel(
        body,
        out_type=jax.ShapeDtypeStruct((32 * 3 * _NPAD,), jnp.float32),
        mesh=mesh,
        compiler_params=_SC_PARAMS,
        scratch_types=[
            pltpu.VMEM((_WSC,), jnp.int32),
            pltpu.VMEM((_WSC,), jnp.float32),
            pltpu.VMEM((_NPAD,), jnp.float32),
        ],
    )(*args)


def _mlp_plane_body(d_ref, a_ref, W1_ref, b1_ref, W2_ref, b2_ref, y_ref, *, din):
    o = [None] * 9
    for k in range(_HID):
        h = d_ref[0] * W1_ref[0, k]
        for j in range(1, din):
            h = h + d_ref[j] * W1_ref[j, k]
        h = jnp.maximum(h + b1_ref[k], 0.0)
        for r in range(9):
            if k == 0:
                o[r] = h * W2_ref[k, r] + b2_ref[r]
            else:
                o[r] = o[r] + h * W2_ref[k, r]
    a0, a1, a2 = a_ref[0], a_ref[1], a_ref[2]
    for i in range(3):
        y_ref[i] = o[3 * i] * a0 + o[3 * i + 1] * a1 + o[3 * i + 2] * a2


def _edge_mlp_planes(dt, at, W1, b1, W2, b2):
    din, r, l = dt.shape
    if l > 128:
        bl = next(c for c in (1280, 1600, 640, 128) if l % c == 0)
        grid = l // bl
        dspec = pl.BlockSpec((din, r, bl), lambda i: (0, 0, i))
        aspec = pl.BlockSpec((3, r, bl), lambda i: (0, 0, i))
    else:
        bl = l
        br = next(c for c in (64, 56, 50, 32, 16, 8, 4) if r % c == 0)
        grid = r // br
        dspec = pl.BlockSpec((din, br, bl), lambda i: (0, i, 0))
        aspec = pl.BlockSpec((3, br, bl), lambda i: (0, i, 0))
    return pl.pallas_call(
        functools.partial(_mlp_plane_body, din=din),
        grid=(grid,),
        in_specs=[
            dspec,
            aspec,
            pl.BlockSpec(memory_space=pltpu.SMEM),
            pl.BlockSpec(memory_space=pltpu.SMEM),
            pl.BlockSpec(memory_space=pltpu.SMEM),
            pl.BlockSpec(memory_space=pltpu.SMEM),
        ],
        out_specs=pl.BlockSpec((3,) + dspec.block_shape[1:],
                               dspec.index_map),
        out_shape=jax.ShapeDtypeStruct((3, r, l), jnp.float32),
    )(dt, at, W1, b1, W2, b2)


def _combine_body(t_ref, o_ref):
    for c in range(3):
        acc = t_ref[c]
        for wk in range(1, 32):
            acc = acc + t_ref[3 * wk + c]
        o_ref[c] = acc


def _combine(partials):
    m = _NPAD // 128
    t = partials.reshape(96, m, 128)
    bm = 56
    return pl.pallas_call(
        _combine_body,
        grid=(m // bm,),
        in_specs=[pl.BlockSpec((96, bm, 128), lambda i: (0, i, 0))],
        out_specs=pl.BlockSpec((3, bm, 128), lambda i: (0, i, 0)),
        out_shape=jax.ShapeDtypeStruct((3, m, 128), jnp.float32),
    )(t)


def kernel(x, edge_2body, edge_3body, edge_2bodySelf, edge_1body,
           edge_attr_2body, edge_attr_3body, edge_attr_2bodySelf, edge_attr_1body,
           W1_2b, b1_2b, W2_2b, b2_2b,
           W1_3b, b1_3b, W2_3b, b2_3b,
           W1_s, b1_s, W2_s, b2_s):
    n = x.shape[0]
    e2 = edge_attr_2body.shape[0]
    e3 = edge_attr_3body.shape[0]
    r2 = next(c for c in (125, 100, 64, 50, 40, 32) if e2 % (c * 128) == 0)
    r3 = next(c for c in (125, 100, 64, 50, 40, 32) if e3 % (c * 128) == 0)

    xtf = x.T.reshape(-1)
    d2f, d3f, dsf = _sc_gather_x(
        xtf, edge_2body[0], edge_2body[1],
        edge_3body[0], edge_3body[1], edge_3body[2],
        edge_2bodySelf[0], edge_2bodySelf[1], e2, e3)
    y2 = _edge_mlp_planes(d2f.reshape(3, r2, e2 // r2),
                          edge_attr_2body.T.reshape(3, r2, e2 // r2),
                          W1_2b, b1_2b, W2_2b, b2_2b)
    y3 = _edge_mlp_planes(d3f.reshape(6, r3, e3 // r3),
                          edge_attr_3body.T.reshape(3, r3, e3 // r3),
                          W1_3b, b1_3b, W2_3b, b2_3b)
    es = edge_attr_2bodySelf.shape[0]
    asp = jnp.pad(edge_attr_2bodySelf.T, ((0, 0), (0, _NPAD - es)))
    ys = _edge_mlp_planes(dsf.reshape(3, _NPAD // 128, 128),
                          asp.reshape(3, _NPAD // 128, 128),
                          W1_s, b1_s, W2_s, b2_s)
    parts = _sc_scatter([
        (y2.reshape(-1), edge_2body[1], e2),
        (y3.reshape(-1), edge_3body[2], e3),
        (ys.reshape(-1), edge_2bodySelf[1], _NPAD),
    ])
    v = _combine(parts).reshape(3, _NPAD)
    return v[:, :n].T


# speedup vs baseline: 28.0984x; 1.0363x over previous
import functools

import jax
import jax.numpy as jnp
from jax import lax
from jax.experimental import pallas as pl
from jax.experimental.pallas import tpu as pltpu
from jax.experimental.pallas import tpu_sc as plsc

_HID = 32
_NPAD = 50176
_WG = 12512
_WSC = 16640

_SC_PARAMS = pltpu.CompilerParams(needs_layout_passes=False)
_MESH = dict(core_axis_name="c", subcore_axis_name="s")


def _wins(per, wmax):
    out, off = [], 0
    while off < per:
        wl = min(wmax, per - off)
        assert wl % 16 == 0, (per, wmax)
        out.append((off, wl))
        off += wl
    return out


def _self_split(e):
    com = (e // 32) // 16 * 16
    rem = e - 32 * com
    assert rem % 16 == 0
    return com, rem


def _sc_gather_x(xt_flat, e2s, e2d, e3j, e3k, e3i, ess, esd, s2, s3):
    n3 = xt_flat.shape[0] // 3
    e2, e3, es = e2s.shape[0], e3j.shape[0], ess.shape[0]
    assert e2 % 32 == 0 and e3 % 32 == 0
    w2, w3 = _wins(e2 // 32, _WG), _wins(e3 // 32, _WG)
    scom, srem = _self_split(es)
    mesh = plsc.VectorSubcoreMesh(**_MESH)

    def body(xt, e2s_r, e2d_r, e3j_r, e3k_r, e3i_r, ess_r, esd_r,
             d2o, d3o, dso, xplane, ib1, ib2, ib3, db1, db2):
        w = lax.axis_index("s") * 2 + lax.axis_index("c")

        def diff_win(base, wlen, src_r, dst_r, out_r, out_off):
            pltpu.sync_copy(src_r.at[pl.ds(base, wlen)],
                            ib1.at[pl.ds(0, wlen)])
            pltpu.sync_copy(dst_r.at[pl.ds(base, wlen)],
                            ib2.at[pl.ds(0, wlen)])

            def g(i, _):
                sl = pl.ds(i * 16, 16)
                db1[sl] = (plsc.load_gather(xplane, [ib1[sl]]) -
                           plsc.load_gather(xplane, [ib2[sl]]))
                return 0

            lax.fori_loop(0, wlen // 16, g, 0, unroll=4)
            pltpu.sync_copy(db1.at[pl.ds(0, wlen)],
                            out_r.at[pl.ds(out_off + base, wlen)])

        def tri_win(base, wlen, c):
            pltpu.sync_copy(e3j_r.at[pl.ds(base, wlen)],
                            ib1.at[pl.ds(0, wlen)])
            pltpu.sync_copy(e3k_r.at[pl.ds(base, wlen)],
                            ib2.at[pl.ds(0, wlen)])
            pltpu.sync_copy(e3i_r.at[pl.ds(base, wlen)],
                            ib3.at[pl.ds(0, wlen)])

            def g(i, _):
                sl = pl.ds(i * 16, 16)
                vj = plsc.load_gather(xplane, [ib1[sl]])
                vk = plsc.load_gather(xplane, [ib2[sl]])
                vi = plsc.load_gather(xplane, [ib3[sl]])
                db1[sl] = vk - vj
                db2[sl] = vi - vk
                return 0

            lax.fori_loop(0, wlen // 16, g, 0, unroll=4)
            pltpu.sync_copy(db1.at[pl.ds(0, wlen)],
                            d3o.at[pl.ds(c * s3 + base, wlen)])
            pltpu.sync_copy(db2.at[pl.ds(0, wlen)],
                            d3o.at[pl.ds((3 + c) * s3 + base, wlen)])

        for c in range(3):
            pltpu.sync_copy(xt.at[pl.ds(c * n3, n3)],
                            xplane.at[pl.ds(0, n3)])
            for off, wl in w2:
                diff_win(w * (e2 // 32) + off, wl, e2s_r, e2d_r, d2o, c * s2)
            for off, wl in w3:
                tri_win(w * (e3 // 32) + off, wl, c)
            diff_win(w * scom, scom, ess_r, esd_r, dso, c * _NPAD)

            @pl.when(w == 0)
            def _(c=c):
                diff_win(32 * scom, srem, ess_r, esd_r, dso, c * _NPAD)

    f32 = jnp.float32
    return pl.kernel(
        body,
        out_type=(
            jax.ShapeDtypeStruct((3 * s2,), f32),
            jax.ShapeDtypeStruct((6 * s3,), f32),
            jax.ShapeDtypeStruct((3 * _NPAD,), f32),
        ),
        mesh=mesh,
        compiler_params=_SC_PARAMS,
        scratch_types=[
            pltpu.VMEM((_NPAD,), f32),
            pltpu.VMEM((_WG,), jnp.int32),
            pltpu.VMEM((_WG,), jnp.int32),
            pltpu.VMEM((_WG,), jnp.int32),
            pltpu.VMEM((_WG,), f32),
            pltpu.VMEM((_WG,), f32),
        ],
    )(xt_flat, e2s, e2d, e3j, e3k, e3i, ess, esd)


def _sc_attr_t(a2f, a3f, asf, s2, s3):
    e2, e3, es = a2f.shape[0] // 3, a3f.shape[0] // 3, asf.shape[0] // 3
    w2, w3 = _wins(e2 // 32, _WG), _wins(e3 // 32, _WG)
    scom, srem = _self_split(es)
    mesh = plsc.VectorSubcoreMesh(**_MESH)

    def body(a2_r, a3_r, as_r, a2o, a3o, aso, abuf, ap0, ap1, ap2):
        w = lax.axis_index("s") * 2 + lax.axis_index("c")
        iota3 = lax.iota(jnp.int32, 16) * 3
        aps = (ap0, ap1, ap2)

        def attr_win(base, wlen, a_r, out_r, stride):
            pltpu.sync_copy(a_r.at[pl.ds(3 * base, 3 * wlen)],
                            abuf.at[pl.ds(0, 3 * wlen)])

            def g(i, _):
                p0 = i * 48 + iota3
                for c in range(3):
                    aps[c][pl.ds(i * 16, 16)] = plsc.load_gather(
                        abuf, [p0 + c])
                return 0

            lax.fori_loop(0, wlen // 16, g, 0, unroll=4)
            for c in range(3):
                pltpu.sync_copy(aps[c].at[pl.ds(0, wlen)],
                                out_r.at[pl.ds(c * stride + base, wlen)])

        for off, wl in w2:
            attr_win(w * (e2 // 32) + off, wl, a2_r, a2o, s2)
        for off, wl in w3:
            attr_win(w * (e3 // 32) + off, wl, a3_r, a3o, s3)
        attr_win(w * scom, scom, as_r, aso, _NPAD)

        @pl.when(w == 0)
        def _():
            attr_win(32 * scom, srem, as_r, aso, _NPAD)

    f32 = jnp.float32
    return pl.kernel(
        body,
        out_type=(
            jax.ShapeDtypeStruct((3 * s2,), f32),
            jax.ShapeDtypeStruct((3 * s3,), f32),
            jax.ShapeDtypeStruct((3 * _NPAD,), f32),
        ),
        mesh=mesh,
        compiler_params=_SC_PARAMS,
        scratch_types=[
            pltpu.VMEM((3 * _WG,), f32),
            pltpu.VMEM((_WG,), f32),
            pltpu.VMEM((_WG,), f32),
            pltpu.VMEM((_WG,), f32),
        ],
    )(a2f, a3f, asf)


def _sc_scatter(streams):
    mesh = plsc.VectorSubcoreMesh(**_MESH)

    def body(*refs):
        stream_refs = refs[:2 * len(streams)]
        out_ref = refs[2 * len(streams)]
        idxb, yb, acc = refs[2 * len(streams) + 1:]
        w = lax.axis_index("s") * 2 + lax.axis_index("c")
        z16 = jnp.zeros((16,), jnp.float32)

        def add_win(base, wlen, y_r, dst_r, c, stride):
            pltpu.sync_copy(dst_r.at[pl.ds(base, wlen)],
                            idxb.at[pl.ds(0, wlen)])
            pltpu.sync_copy(y_r.at[pl.ds(c * stride + base, wlen)],
                            yb.at[pl.ds(0, wlen)])

            def g(i, _):
                sl = pl.ds(i * 16, 16)
                plsc.addupdate_scatter(acc, [idxb[sl]], yb[sl])
                return 0

            lax.fori_loop(0, wlen // 16, g, 0, unroll=8)

        for c in range(3):
            def zero_body(i, _):
                acc[pl.ds(i * 16, 16)] = z16
                return 0

            lax.fori_loop(0, _NPAD // 16, zero_body, 0, unroll=8)
            for si in range(len(streams)):
                y_r = stream_refs[2 * si]
                dst_r = stream_refs[2 * si + 1]
                e = streams[si][1].shape[0]
                stride = streams[si][2]
                if e % 32 == 0 and (e // 32) % 16 == 0:
                    for off, wl in _wins(e // 32, _WSC):
                        add_win(w * (e // 32) + off, wl, y_r, dst_r, c,
                                stride)
                else:
                    scom, srem = _self_split(e)
                    add_win(w * scom, scom, y_r, dst_r, c, stride)

                    @pl.when(w == 0)
                    def _(y_r=y_r, dst_r=dst_r, c=c, stride=stride,
                          scom=scom, srem=srem):
                        add_win(32 * scom, srem, y_r, dst_r, c, stride)
            pltpu.sync_copy(acc,
                            out_ref.at[pl.ds((w * 3 + c) * _NPAD, _NPAD)])

    args = []
    for y, dst, stride in streams:
        args += [y, dst]
    return pl.kernel(
        body,
        out_type=jax.ShapeDtypeStruct((32 * 3 * _NPAD,), jnp.float32),
        mesh=mesh,
        compiler_params=_SC_PARAMS,
        scratch_types=[
            pltpu.VMEM((_WSC,), jnp.int32),
            pltpu.VMEM((_WSC,), jnp.float32),
            pltpu.VMEM((_NPAD,), jnp.float32),
        ],
    )(*args)


def _mlp_plane_body(d_ref, a_ref, W1_ref, b1_ref, W2_ref, b2_ref, y_ref, *, din):
    o = [None] * 9
    for k in range(_HID):
        h = d_ref[0] * W1_ref[0, k]
        for j in range(1, din):
            h = h + d_ref[j] * W1_ref[j, k]
        h = jnp.maximum(h + b1_ref[k], 0.0)
        for r in range(9):
            if k == 0:
                o[r] = h * W2_ref[k, r] + b2_ref[r]
            else:
                o[r] = o[r] + h * W2_ref[k, r]
    a0, a1, a2 = a_ref[0], a_ref[1], a_ref[2]
    for i in range(3):
        y_ref[i] = o[3 * i] * a0 + o[3 * i + 1] * a1 + o[3 * i + 2] * a2


def _edge_mlp_planes(dt, at, W1, b1, W2, b2):
    din, r, l = dt.shape
    if l > 128:
        bl = next(c for c in (1280, 1600, 640, 128) if l % c == 0)
        grid = l // bl
        dspec = pl.BlockSpec((din, r, bl), lambda i: (0, 0, i))
        aspec = pl.BlockSpec((3, r, bl), lambda i: (0, 0, i))
    else:
        bl = l
        br = next(c for c in (64, 56, 50, 32, 16, 8, 4) if r % c == 0)
        grid = r // br
        dspec = pl.BlockSpec((din, br, bl), lambda i: (0, i, 0))
        aspec = pl.BlockSpec((3, br, bl), lambda i: (0, i, 0))
    return pl.pallas_call(
        functools.partial(_mlp_plane_body, din=din),
        grid=(grid,),
        in_specs=[
            dspec,
            aspec,
            pl.BlockSpec(memory_space=pltpu.SMEM),
            pl.BlockSpec(memory_space=pltpu.SMEM),
            pl.BlockSpec(memory_space=pltpu.SMEM),
            pl.BlockSpec(memory_space=pltpu.SMEM),
        ],
        out_specs=pl.BlockSpec((3,) + dspec.block_shape[1:],
                               dspec.index_map),
        out_shape=jax.ShapeDtypeStruct((3, r, l), jnp.float32),
    )(dt, at, W1, b1, W2, b2)


def _combine_body(t_ref, o_ref):
    for c in range(3):
        acc = t_ref[c]
        for wk in range(1, 32):
            acc = acc + t_ref[3 * wk + c]
        o_ref[c] = acc


def _combine(partials):
    m = _NPAD // 128
    t = partials.reshape(96, m, 128)
    bm = 56
    return pl.pallas_call(
        _combine_body,
        grid=(m // bm,),
        in_specs=[pl.BlockSpec((96, bm, 128), lambda i: (0, i, 0))],
        out_specs=pl.BlockSpec((3, bm, 128), lambda i: (0, i, 0)),
        out_shape=jax.ShapeDtypeStruct((3, m, 128), jnp.float32),
    )(t)


def kernel(x, edge_2body, edge_3body, edge_2bodySelf, edge_1body,
           edge_attr_2body, edge_attr_3body, edge_attr_2bodySelf, edge_attr_1body,
           W1_2b, b1_2b, W2_2b, b2_2b,
           W1_3b, b1_3b, W2_3b, b2_3b,
           W1_s, b1_s, W2_s, b2_s):
    n = x.shape[0]
    e2 = edge_attr_2body.shape[0]
    e3 = edge_attr_3body.shape[0]
    r2 = next(c for c in (125, 100, 64, 50, 40, 32) if e2 % (c * 128) == 0)
    r3 = next(c for c in (125, 100, 64, 50, 40, 32) if e3 % (c * 128) == 0)

    xtf = x.T.reshape(-1)
    d2f, d3f, dsf = _sc_gather_x(
        xtf, edge_2body[0], edge_2body[1],
        edge_3body[0], edge_3body[1], edge_3body[2],
        edge_2bodySelf[0], edge_2bodySelf[1], e2, e3)
    y2 = _edge_mlp_planes(d2f.reshape(3, r2, e2 // r2),
                          edge_attr_2body.T.reshape(3, r2, e2 // r2),
                          W1_2b, b1_2b, W2_2b, b2_2b)
    y3 = _edge_mlp_planes(d3f.reshape(6, r3, e3 // r3),
                          edge_attr_3body.T.reshape(3, r3, e3 // r3),
                          W1_3b, b1_3b, W2_3b, b2_3b)
    es = edge_attr_2bodySelf.shape[0]
    asp = jnp.pad(edge_attr_2bodySelf.T, ((0, 0), (0, _NPAD - es)))
    ys = _edge_mlp_planes(dsf.reshape(3, _NPAD // 128, 128),
                          asp.reshape(3, _NPAD // 128, 128),
                          W1_s, b1_s, W2_s, b2_s)
    parts = _sc_scatter([
        (y2.reshape(-1), edge_2body[1], e2),
        (y3.reshape(-1), edge_3body[2], e3),
        (ys.reshape(-1), edge_2bodySelf[1], _NPAD),
    ])
    v = _combine(parts).reshape(3, _NPAD)
    return v[:, :n].T
